# baseline (speedup 1.0000x reference)
import functools

import jax
import jax.numpy as jnp
from jax import lax
from jax.experimental import pallas as pl
from jax.experimental.pallas import tpu as pltpu
from jax.experimental.pallas import tpu_sc as plsc

SCALE = 8.0
NW = 32
LANES = 16
NBUF = 4
C = 128
C0 = 320


def _compact_table(table):
    V, D = table.shape
    NCH = V // C0
    KMAX = (NCH + NW - 1) // NW

    mesh = plsc.VectorSubcoreMesh(core_axis_name="c", subcore_axis_name="s")

    @functools.partial(
        pl.kernel,
        mesh=mesh,
        out_type=jax.ShapeDtypeStruct((V // 2, 2 * D), jnp.float32),
        scratch_types=[
            [pltpu.VMEM((C0, D), jnp.float32) for _ in range(2)],
            [pltpu.VMEM((C0 // 2, 2 * D), jnp.float32) for _ in range(2)],
            [pltpu.SemaphoreType.DMA for _ in range(2)],
            [pltpu.SemaphoreType.DMA for _ in range(2)],
        ],
        compiler_params=pltpu.CompilerParams(use_tc_tiling_on_sc=True),
    )
    def compact(tbl_hbm, out_hbm, ibufs, obufs, rsems, wsems):
        wid = lax.axis_index("s") * 2 + lax.axis_index("c")

        def fire(ch, b):
            pltpu.async_copy(
                tbl_hbm.at[pl.ds(ch * C0, C0)], ibufs[b], rsems[b])

        def rdrain(b):
            pltpu.make_async_copy(
                tbl_hbm.at[pl.ds(0, C0)], ibufs[b], rsems[b]).wait()

        def wdrain(b):
            pltpu.make_async_copy(
                out_hbm.at[pl.ds(0, C0 // 2)], obufs[b], wsems[b]).wait()

        def squeeze(b):
            def pair_body(p, carry):
                for h in range(2):
                    r = 2 * p + h
                    for s in range(D // LANES):
                        src = pl.ds(s * LANES, LANES)
                        dst = pl.ds(h * D + s * LANES, LANES)
                        obufs[b][p, dst] = ibufs[b][r, src] * SCALE
                return carry
            lax.fori_loop(0, C0 // 2, pair_body, 0)

        @pl.when(wid < NCH)
        def _():
            fire(wid, 0)

        def body(k, carry):
            for b in range(2):
                kk = k * 2 + b
                ch = wid + NW * kk
                nch = ch + NW

                @pl.when(nch < NCH)
                def _():
                    fire(nch, 1 - b)

                @pl.when(ch < NCH)
                def _():
                    rdrain(b)

                    @pl.when(kk >= 2)
                    def _():
                        wdrain(b)

                    squeeze(b)
                    pltpu.async_copy(
                        obufs[b],
                        out_hbm.at[pl.ds(ch * (C0 // 2), C0 // 2)],
                        wsems[b])
            return carry

        lax.fori_loop(0, (KMAX + 1) // 2, body, 0)
        nkk = (NCH - 1 - wid) // NW + 1
        for b in range(2):
            @pl.when(nkk > b)
            def _():
                wdrain(b)

    return compact(table)


def kernel(input, table):
    R, S = input.shape
    B = R * S
    V, D = table.shape
    BW = B // NW
    NCHUNK = BW // C

    idx = input.reshape(B // C, C)
    tbl = _compact_table(table).reshape(V, D)

    mesh = plsc.VectorSubcoreMesh(core_axis_name="c", subcore_axis_name="s")

    @functools.partial(
        pl.kernel,
        mesh=mesh,
        out_type=jax.ShapeDtypeStruct((B, 2 * D), jnp.float32),
        scratch_types=[
            pltpu.VMEM((NCHUNK, C), jnp.int32),
            [pltpu.VMEM((C, D), jnp.float32) for _ in range(NBUF)],
            [pltpu.SemaphoreType.DMA for _ in range(NBUF)],
            [pltpu.SemaphoreType.DMA for _ in range(NBUF)],
        ],
        compiler_params=pltpu.CompilerParams(use_tc_tiling_on_sc=False),
    )
    def emb(idx_hbm, table_hbm, out_hbm, idx_v, gbufs, gsems, ssems):
        wid = lax.axis_index("s") * 2 + lax.axis_index("c")
        base = wid * BW
        pltpu.sync_copy(idx_hbm.at[pl.ds(wid * NCHUNK, NCHUNK)], idx_v)

        def fire(c, t):
            pltpu.async_copy(table_hbm.at[idx_v.at[c]], gbufs[t], gsems[t])

        def gdrain(t):
            pltpu.make_async_copy(
                table_hbm.at[pl.ds(0, C)], gbufs[t], gsems[t]).wait()

        def sdrain(t):
            pltpu.make_async_copy(
                out_hbm.at[pl.ds(0, C), pl.ds(0, D)], gbufs[t],
                ssems[t]).wait()

        for t in range(NBUF - 1):
            fire(t, t)

        def body(i, carry):
            for t in range(NBUF):
                c = i * NBUF + t
                gdrain(t)
                pltpu.async_copy(
                    gbufs[t],
                    out_hbm.at[pl.ds(base + c * C, C), pl.ds(0, D)],
                    ssems[t])
                nt = (t + NBUF - 1) % NBUF
                nc = c + NBUF - 1

                @pl.when(jnp.logical_and(c >= 1, nc <= NCHUNK - 1))
                def _():
                    sdrain(nt)

                @pl.when(nc <= NCHUNK - 1)
                def _():
                    fire(nc, nt)
            return carry

        lax.fori_loop(0, NCHUNK // NBUF, body, 0)
        for t in range(NBUF):
            sdrain(t)

    out = emb(idx, tbl)
    return out[:, :D].reshape(R, S, D)


# speedup vs baseline: 1.3283x; 1.3283x over previous
import functools

import jax
import jax.numpy as jnp
from jax import lax
from jax.experimental import pallas as pl
from jax.experimental.pallas import tpu as pltpu
from jax.experimental.pallas import tpu_sc as plsc

SCALE = 8.0
NW = 32
LANES = 16
NBUF = 4
C = 128


def kernel(input, table):
    R, S = input.shape
    B = R * S
    V, D = table.shape
    BW = B // NW
    NCHUNK = BW // C

    idx = input.reshape(B // C, C)

    mesh = plsc.VectorSubcoreMesh(core_axis_name="c", subcore_axis_name="s")

    @functools.partial(
        pl.kernel,
        mesh=mesh,
        out_type=jax.ShapeDtypeStruct((B, 2 * D), jnp.float32),
        scratch_types=[
            pltpu.VMEM((NCHUNK, C), jnp.int32),
            [pltpu.VMEM((C, D), jnp.float32) for _ in range(NBUF)],
            [pltpu.SemaphoreType.DMA for _ in range(NBUF)],
            [pltpu.SemaphoreType.DMA for _ in range(NBUF)],
        ],
        compiler_params=pltpu.CompilerParams(use_tc_tiling_on_sc=False),
    )
    def emb(idx_hbm, table_hbm, out_hbm, idx_v, gbufs, gsems, ssems):
        wid = lax.axis_index("s") * 2 + lax.axis_index("c")
        base = wid * BW
        pltpu.sync_copy(idx_hbm.at[pl.ds(wid * NCHUNK, NCHUNK)], idx_v)

        def fire(c, t):
            pltpu.async_copy(table_hbm.at[idx_v.at[c]], gbufs[t], gsems[t])

        def gdrain(t):
            pltpu.make_async_copy(
                table_hbm.at[pl.ds(0, C)], gbufs[t], gsems[t]).wait()

        def sdrain(t):
            pltpu.make_async_copy(
                out_hbm.at[pl.ds(0, C), pl.ds(0, D)], gbufs[t],
                ssems[t]).wait()

        def process(t):
            def row_body(r, carry):
                for s in range(D // LANES):
                    sl = pl.ds(s * LANES, LANES)
                    gbufs[t][r, sl] = gbufs[t][r, sl] * SCALE
                return carry
            lax.fori_loop(0, C, row_body, 0)

        for t in range(NBUF - 1):
            fire(t, t)

        def body(i, carry):
            for t in range(NBUF):
                c = i * NBUF + t
                gdrain(t)
                process(t)
                pltpu.async_copy(
                    gbufs[t],
                    out_hbm.at[pl.ds(base + c * C, C), pl.ds(0, D)],
                    ssems[t])
                nt = (t + NBUF - 1) % NBUF
                nc = c + NBUF - 1

                @pl.when(jnp.logical_and(c >= 1, nc <= NCHUNK - 1))
                def _():
                    sdrain(nt)

                @pl.when(nc <= NCHUNK - 1)
                def _():
                    fire(nc, nt)
            return carry

        lax.fori_loop(0, NCHUNK // NBUF, body, 0)
        for t in range(NBUF):
            sdrain(t)

    out = emb(idx, table)
    return out[:, :D].reshape(R, S, D)
